# baseline (speedup 1.0000x reference)
import functools

import jax
import jax.numpy as jnp
from jax import lax
from jax.experimental import pallas as pl
from jax.experimental.pallas import tpu as pltpu
from jax.experimental.pallas import tpu_sc as plsc

MASKS_SIZE = 64
GRID_W = 6
C_CHUNK = 24
MW = 8 * 384


@functools.partial(
    pl.kernel,
    out_type=jax.ShapeDtypeStruct((8, MW), jnp.float32),
    mesh=plsc.VectorSubcoreMesh(core_axis_name="c", subcore_axis_name="s"),
    scratch_types=[
        pltpu.VMEM((MW,), jnp.float32),
        pltpu.VMEM((16,), jnp.int32),
    ],
)
def _sc_mask_build(mi_hbm, out_hbm, m_v, idx_v):
    wid = lax.axis_index("s") * 2 + lax.axis_index("c")

    @pl.when(wid < 8)
    def _():
        pltpu.sync_copy(mi_hbm.at[wid], idx_v)
        idx_vec = idx_v[...]
        ones = jnp.ones((16,), jnp.float32)
        for j in range(MW // 16):
            m_v[pl.ds(j * 16, 16)] = ones
        zeros = jnp.zeros((16,), jnp.float32)
        for k in range(4):
            cell = idx_vec[k]
            base = (cell // GRID_W) * 384 + (cell % GRID_W) * MASKS_SIZE
            for j in range(MASKS_SIZE // 16):
                m_v[pl.ds(base + j * 16, 16)] = zeros
        pltpu.sync_copy(m_v, out_hbm.at[wid])


def _mul_body(m_ref, x_ref, o_ref):
    for r in range(6):
        band = slice(r * MASKS_SIZE, (r + 1) * MASKS_SIZE)
        o_ref[0, :, band, :] = x_ref[0, :, band, :] * m_ref[0, r, :][None, None, :]


def kernel(x, m_indices):
    b, c, h, w = x.shape
    mi2 = jnp.tile(m_indices, (1, 4))
    mask_rows = _sc_mask_build(mi2).reshape(b, 8, w)
    grid = (b, c // C_CHUNK)
    return pl.pallas_call(
        _mul_body,
        grid=grid,
        in_specs=[
            pl.BlockSpec((1, 8, w), lambda i, j: (i, 0, 0)),
            pl.BlockSpec((1, C_CHUNK, h, w), lambda i, j: (i, j, 0, 0)),
        ],
        out_specs=pl.BlockSpec((1, C_CHUNK, h, w), lambda i, j: (i, j, 0, 0)),
        out_shape=jax.ShapeDtypeStruct(x.shape, x.dtype),
    )(mask_rows, x)
